# baseline (speedup 1.0000x reference)
import functools

import jax
import jax.numpy as jnp
from jax import lax
from jax.experimental import pallas as pl
from jax.experimental.pallas import tpu as pltpu
from jax.experimental.pallas import tpu_sc as plsc

N = 10000
K = 16
D = 8
S = 2000
E = 320000

SP = 2048
TB = 512
NSC = 32
CH = E // NSC
NT = 2 * K + 1

_HI = jax.lax.Precision.HIGHEST

_SC_PARAMS = dict(
    compiler_params=pltpu.CompilerParams(use_tc_tiling_on_sc=False,
                                         needs_layout_passes=False),
)


_HF = SP // 2


def _sgather_body(z_hbm, gate_hbm, b_hbm, sidx_hbm, sg_hbm, gs_hbm,
                  row_v, idx_v, out_v, idx64_v, rows_v, sem):
    wid = lax.axis_index("s") * 2 + lax.axis_index("c")
    cp_idx = pltpu.async_copy(sidx_hbm, idx_v, sem)

    pltpu.sync_copy(sidx_hbm.at[pl.ds(wid * 64, 64)], idx64_v)
    pltpu.async_copy(gate_hbm.at[idx64_v], rows_v, sem).wait()
    pltpu.sync_copy(rows_v, gs_hbm.at[pl.ds(wid * 64, 64)])

    zrow = wid // 2
    half = wid % 2
    pltpu.sync_copy(z_hbm.at[zrow], row_v)
    cp_idx.wait()

    def body(i, _):
        iv = idx_v[pl.ds(half * _HF + i * 16, 16)]
        out_v[pl.ds(i * 16, 16)] = plsc.load_gather(row_v, [iv])
        return 0

    lax.fori_loop(0, _HF // 16, body, 0)
    pltpu.sync_copy(out_v, sg_hbm.at[zrow, pl.ds(half * _HF, _HF)])

    @pl.when(wid < 2)
    def _():
        pltpu.sync_copy(b_hbm, row_v)

        def body2(i, _):
            iv = idx_v[pl.ds(wid * _HF + i * 16, 16)]
            out_v[pl.ds(i * 16, 16)] = plsc.load_gather(row_v, [iv])
            return 0

        lax.fori_loop(0, _HF // 16, body2, 0)
        pltpu.sync_copy(out_v, sg_hbm.at[K, pl.ds(wid * _HF, _HF)])


@functools.cache
def _sgather_call():
    return functools.partial(
        pl.kernel,
        out_type=[
            jax.ShapeDtypeStruct((K + 1, SP), jnp.float32),
            jax.ShapeDtypeStruct((SP, K), jnp.float32),
        ],
        mesh=plsc.VectorSubcoreMesh(core_axis_name="c", subcore_axis_name="s"),
        scratch_types=[
            pltpu.VMEM((N,), jnp.float32),
            pltpu.VMEM((SP,), jnp.int32),
            pltpu.VMEM((_HF,), jnp.float32),
            pltpu.VMEM((64,), jnp.int32),
            pltpu.VMEM((64, K), jnp.float32),
            pltpu.SemaphoreType.DMA,
        ],
        **_SC_PARAMS,
    )(_sgather_body)


def _pack_rows(p):
    pb = p.astype(jnp.bfloat16)
    lo = lax.convert_element_type(
        lax.bitcast_convert_type(pb[0:D // 2, :], jnp.uint16), jnp.uint32)
    hi = lax.convert_element_type(
        lax.bitcast_convert_type(pb[D // 2:D, :], jnp.uint16), jnp.uint32)
    return lax.bitcast_convert_type(lo | (hi << 16), jnp.int32)


def _dense_body(z_ref, gate_ref, a_ref, sg_ref, gsamp_ref,
                x_ref, bsm_ref, p_ref):
    z = z_ref[...]
    zmax = jnp.max(z, axis=0, keepdims=True)
    ez = jnp.exp(z - zmax)
    zs = ez / jnp.sum(ez, axis=0, keepdims=True)
    g = 1.0 / (1.0 + jnp.exp(-gate_ref[...]))
    mm = lax.dot_general(zs, g, (((1,), (0,)), ((), ())),
                         preferred_element_type=jnp.float32, precision=_HI)
    eye = jnp.eye(K, dtype=jnp.float32)
    colsum = jnp.sum(mm * eye, axis=0, keepdims=True)

    sg = sg_ref[...]
    zraw = sg[0:K, :]
    braw = sg[K:K + 1, :]
    zm = jnp.max(zraw, axis=0, keepdims=True)
    es = jnp.exp(zraw - zm)
    zsamp = es / jnp.sum(es, axis=0, keepdims=True)
    gs = 1.0 / (1.0 + jnp.exp(-lax.transpose(gsamp_ref[...], (1, 0))))

    lane = lax.broadcasted_iota(jnp.int32, (1, SP), 1)
    valid = lane < S
    ztgsampt = jnp.where(valid, zsamp * gs, 0.0)
    bsm_ref[...] = jnp.where(valid, braw, -1e30)

    m = lax.dot_general(zsamp, ztgsampt, (((1,), (1,)), ((), ())),
                        preferred_element_type=jnp.float32, precision=_HI)
    m = m / colsum
    azc = lax.dot_general(a_ref[...], m, (((1,), (0,)), ((), ())),
                          preferred_element_type=jnp.float32, precision=_HI)
    x_ref[...] = lax.dot_general(azc, zsamp, (((1,), (0,)), ((), ())),
                                 preferred_element_type=jnp.float32,
                                 precision=_HI)
    p = lax.dot_general(azc, zs, (((1,), (0,)), ((), ())),
                        preferred_element_type=jnp.float32, precision=_HI)
    p_ref[...] = _pack_rows(p)


_dense_call = pl.pallas_call(
    _dense_body,
    out_shape=[
        jax.ShapeDtypeStruct((D, SP), jnp.float32),
        jax.ShapeDtypeStruct((1, SP), jnp.float32),
        jax.ShapeDtypeStruct((D // 2, N), jnp.int32),
    ],
)


def _pair_body(x_ref, bsm_ref, xt_ref, bst_ref, z1_ref):
    s = pl.program_id(0)
    x = x_ref[...]
    bs = bsm_ref[...]
    xi = xt_ref[...]
    bst = bst_ref[...]

    ycol = xi * (xi + 2e-6)
    ccol = lax.dot_general(ycol, jnp.ones((D, 1), jnp.float32),
                           (((0,), (0,)), ((), ())),
                           preferred_element_type=jnp.float32, precision=_HI)
    bcol = lax.dot_general(bst, jnp.ones((1, 1), jnp.float32),
                           (((0,), (0,)), ((), ())),
                           preferred_element_type=jnp.float32, precision=_HI)
    yrow = x * (x - 2e-6)
    rrow = lax.dot_general(jnp.ones((1, D), jnp.float32), yrow,
                           (((1,), (0,)), ((), ())),
                           preferred_element_type=jnp.float32, precision=_HI)
    cross = lax.dot_general(xi, x, (((0,), (0,)), ((), ())),
                            preferred_element_type=jnp.float32)
    d2 = jnp.maximum(ccol + rrow - 2.0 * cross + (D * 1e-12), 1e-30)
    dist = d2 * lax.rsqrt(d2)
    mat = jnp.exp(bcol + bs - dist)
    part = 0.5 * jnp.sum(mat, keepdims=True)

    @pl.when(s == 0)
    def _():
        diag = jnp.exp(2.0 * bs - (D ** 0.5) * 1e-6)
        z1_ref[...] = -0.5 * jnp.sum(diag, keepdims=True)

    z1_ref[...] += part


_pair_call = pl.pallas_call(
    _pair_body,
    grid=(SP // TB,),
    in_specs=[
        pl.BlockSpec((D, SP), lambda s: (0, 0)),
        pl.BlockSpec((1, SP), lambda s: (0, 0)),
        pl.BlockSpec((D, TB), lambda s: (0, s)),
        pl.BlockSpec((1, TB), lambda s: (0, s)),
    ],
    out_specs=[
        pl.BlockSpec((1, 1), lambda s: (0, 0)),
    ],
    out_shape=[
        jax.ShapeDtypeStruct((1, 1), jnp.float32),
    ],
)


def _edge_body(p_hbm, beta_hbm, ii_hbm, jj_hbm, vc_hbm, out_hbm,
               p_v, b_v, ii_v, jj_v, vc_v, acc_v, sem):
    wid = lax.axis_index("s") * 2 + lax.axis_index("c")
    base = wid * CH
    copies = [
        pltpu.async_copy(p_hbm, p_v, sem),
        pltpu.async_copy(beta_hbm, b_v, sem),
        pltpu.async_copy(ii_hbm.at[pl.ds(base, CH)], ii_v, sem),
        pltpu.async_copy(jj_hbm.at[pl.ds(base, CH)], jj_v, sem),
        pltpu.async_copy(vc_hbm.at[pl.ds(base, CH)], vc_v, sem),
    ]
    for c in copies:
        c.wait()

    eps = jnp.full((32,), 1e-6, jnp.bfloat16)

    def body(t, acc):
        off = t * 16
        iv = ii_v[pl.ds(off, 16)]
        jv = jj_v[pl.ds(off, 16)]
        vc = vc_v[pl.ds(off, 16)]
        bi = plsc.load_gather(b_v, [iv])
        bj = plsc.load_gather(b_v, [jv])
        sqb = jnp.zeros((32,), jnp.bfloat16)
        for r in range(D // 2):
            row = jnp.full((16,), r, jnp.int32)
            wi = plsc.bitcast(plsc.load_gather(p_v, [row, iv]), jnp.bfloat16)
            wj = plsc.bitcast(plsc.load_gather(p_v, [row, jv]), jnp.bfloat16)
            t0 = wi - wj + eps
            sqb = sqb + t0 * t0
        slo, shi = plsc.unpack(sqb, format=plsc.PackFormat.INTERLEAVED)
        xc = jnp.maximum(slo + shi, 1e-30)
        yi = jnp.int32(0x5F3759DF) - (plsc.bitcast(xc, jnp.int32) >> 1)
        y = plsc.bitcast(yi, jnp.float32)
        y = y * (1.5 - 0.5 * xc * y * y)
        dist = xc * y
        return acc + vc * ((bi + bj) - dist)

    acc = lax.fori_loop(0, CH // 16, body, jnp.zeros((16,), jnp.float32),
                        unroll=2)
    acc_v[...] = acc
    pltpu.sync_copy(acc_v, out_hbm.at[wid])


@functools.cache
def _edge_call():
    return functools.partial(
        pl.kernel,
        out_type=jax.ShapeDtypeStruct((NSC, 16), jnp.float32),
        mesh=plsc.VectorSubcoreMesh(core_axis_name="c", subcore_axis_name="s"),
        scratch_types=[
            pltpu.VMEM((D // 2, N), jnp.int32),
            pltpu.VMEM((N,), jnp.float32),
            pltpu.VMEM((CH,), jnp.int32),
            pltpu.VMEM((CH,), jnp.int32),
            pltpu.VMEM((CH,), jnp.float32),
            pltpu.VMEM((16,), jnp.float32),
            pltpu.SemaphoreType.DMA,
        ],
        **_SC_PARAMS,
    )(_edge_body)


def kernel(beta, A, Z, Gate, sample_idx, sparse_sample_i, sparse_sample_j,
           valueC):
    zf = Z.astype(jnp.float32)
    gf = Gate.astype(jnp.float32)
    bf = beta.astype(jnp.float32)
    sidxp = jnp.pad(sample_idx.astype(jnp.int32), (0, SP - S))

    sg, gsamp = _sgather_call()(zf, gf, bf, sidxp)
    x, bsm, p = _dense_call(zf, gf, A.astype(jnp.float32), sg, gsamp)
    partials = _edge_call()(p, bf, sparse_sample_i.astype(jnp.int32),
                            sparse_sample_j.astype(jnp.int32),
                            valueC.astype(jnp.float32))
    (z1,) = _pair_call(x, bsm, x, bsm)
    return jnp.sum(partials) - z1[0, 0]


# speedup vs baseline: 1.0561x; 1.0561x over previous
import functools

import jax
import jax.numpy as jnp
from jax import lax
from jax.experimental import pallas as pl
from jax.experimental.pallas import tpu as pltpu
from jax.experimental.pallas import tpu_sc as plsc

N = 10000
K = 16
D = 8
S = 2000
E = 320000

SP = 2048
TB = 512
NSC = 32
CH = E // NSC
NT = 2 * K + 1

_HI = jax.lax.Precision.HIGHEST

_SC_PARAMS = dict(
    compiler_params=pltpu.CompilerParams(use_tc_tiling_on_sc=False,
                                         needs_layout_passes=False),
)


_HF = SP // 2


def _sgather_body(z_hbm, gate_hbm, b_hbm, sidx_hbm, sg_hbm, gs_hbm,
                  row_v, idx_v, out_v, idx64_v, rows_v, sem):
    wid = lax.axis_index("s") * 2 + lax.axis_index("c")
    cp_idx = pltpu.async_copy(sidx_hbm, idx_v, sem)

    pltpu.sync_copy(sidx_hbm.at[pl.ds(wid * 64, 64)], idx64_v)
    pltpu.async_copy(gate_hbm.at[idx64_v], rows_v, sem).wait()
    pltpu.sync_copy(rows_v, gs_hbm.at[pl.ds(wid * 64, 64)])

    zrow = wid // 2
    half = wid % 2
    pltpu.sync_copy(z_hbm.at[zrow], row_v)
    cp_idx.wait()

    def body(i, _):
        iv = idx_v[pl.ds(half * _HF + i * 16, 16)]
        out_v[pl.ds(i * 16, 16)] = plsc.load_gather(row_v, [iv])
        return 0

    lax.fori_loop(0, _HF // 16, body, 0)
    pltpu.sync_copy(out_v, sg_hbm.at[zrow, pl.ds(half * _HF, _HF)])

    @pl.when(wid < 2)
    def _():
        pltpu.sync_copy(b_hbm, row_v)

        def body2(i, _):
            iv = idx_v[pl.ds(wid * _HF + i * 16, 16)]
            out_v[pl.ds(i * 16, 16)] = plsc.load_gather(row_v, [iv])
            return 0

        lax.fori_loop(0, _HF // 16, body2, 0)
        pltpu.sync_copy(out_v, sg_hbm.at[K, pl.ds(wid * _HF, _HF)])


@functools.cache
def _sgather_call():
    return functools.partial(
        pl.kernel,
        out_type=[
            jax.ShapeDtypeStruct((K + 1, SP), jnp.float32),
            jax.ShapeDtypeStruct((SP, K), jnp.float32),
        ],
        mesh=plsc.VectorSubcoreMesh(core_axis_name="c", subcore_axis_name="s"),
        scratch_types=[
            pltpu.VMEM((N,), jnp.float32),
            pltpu.VMEM((SP,), jnp.int32),
            pltpu.VMEM((_HF,), jnp.float32),
            pltpu.VMEM((64,), jnp.int32),
            pltpu.VMEM((64, K), jnp.float32),
            pltpu.SemaphoreType.DMA,
        ],
        **_SC_PARAMS,
    )(_sgather_body)


def _pack_rows(p):
    pb = p.astype(jnp.bfloat16)
    lo = lax.convert_element_type(
        lax.bitcast_convert_type(pb[0:D // 2, :], jnp.uint16), jnp.uint32)
    hi = lax.convert_element_type(
        lax.bitcast_convert_type(pb[D // 2:D, :], jnp.uint16), jnp.uint32)
    return lax.bitcast_convert_type(lo | (hi << 16), jnp.int32)


def _dense_body(z_ref, gt_ref, a_ref, sg_ref, gsamp_ref,
                x_ref, bsm_ref, p_ref):
    z = z_ref[...]
    zmax = jnp.max(z, axis=0, keepdims=True)
    ez = jnp.exp(z - zmax)
    zs = ez / jnp.sum(ez, axis=0, keepdims=True)
    g = 1.0 / (1.0 + jnp.exp(-gt_ref[...]))
    ztgt = zs * g
    ones_row = jnp.ones((1, N), jnp.float32)
    colsum = lax.dot_general(ones_row, ztgt, (((1,), (1,)), ((), ())),
                             preferred_element_type=jnp.float32, precision=_HI)

    sg = sg_ref[...]
    zraw = sg[0:K, :]
    braw = sg[K:K + 1, :]
    zm = jnp.max(zraw, axis=0, keepdims=True)
    es = jnp.exp(zraw - zm)
    zsamp = es / jnp.sum(es, axis=0, keepdims=True)
    gs = 1.0 / (1.0 + jnp.exp(-lax.transpose(gsamp_ref[...], (1, 0))))

    lane = lax.broadcasted_iota(jnp.int32, (1, SP), 1)
    valid = lane < S
    ztgsampt = jnp.where(valid, zsamp * gs, 0.0)
    bsm_ref[...] = jnp.where(valid, braw, -1e30)

    m = lax.dot_general(zsamp, ztgsampt, (((1,), (1,)), ((), ())),
                        preferred_element_type=jnp.float32, precision=_HI)
    m = m / colsum
    azc = lax.dot_general(a_ref[...], m, (((1,), (0,)), ((), ())),
                          preferred_element_type=jnp.float32, precision=_HI)
    x_ref[...] = lax.dot_general(azc, zsamp, (((1,), (0,)), ((), ())),
                                 preferred_element_type=jnp.float32,
                                 precision=_HI)
    p = lax.dot_general(azc, zs, (((1,), (0,)), ((), ())),
                        preferred_element_type=jnp.float32, precision=_HI)
    p_ref[...] = _pack_rows(p)


_dense_call = pl.pallas_call(
    _dense_body,
    out_shape=[
        jax.ShapeDtypeStruct((D, SP), jnp.float32),
        jax.ShapeDtypeStruct((1, SP), jnp.float32),
        jax.ShapeDtypeStruct((D // 2, N), jnp.int32),
    ],
)


def _pair_body(x_ref, bsm_ref, xt_ref, bst_ref, z1_ref):
    s = pl.program_id(0)
    x = x_ref[...]
    bs = bsm_ref[...]
    xi = xt_ref[...]
    bst = bst_ref[...]

    ycol = xi * (xi + 2e-6)
    ccol = lax.dot_general(ycol, jnp.ones((D, 1), jnp.float32),
                           (((0,), (0,)), ((), ())),
                           preferred_element_type=jnp.float32, precision=_HI)
    bcol = lax.dot_general(bst, jnp.ones((1, 1), jnp.float32),
                           (((0,), (0,)), ((), ())),
                           preferred_element_type=jnp.float32, precision=_HI)
    yrow = x * (x - 2e-6)
    rrow = lax.dot_general(jnp.ones((1, D), jnp.float32), yrow,
                           (((1,), (0,)), ((), ())),
                           preferred_element_type=jnp.float32, precision=_HI)
    cross = lax.dot_general(xi, x, (((0,), (0,)), ((), ())),
                            preferred_element_type=jnp.float32)
    d2 = jnp.maximum(ccol + rrow - 2.0 * cross + (D * 1e-12), 1e-30)
    dist = d2 * lax.rsqrt(d2)
    mat = jnp.exp(bcol + bs - dist)
    part = 0.5 * jnp.sum(mat, keepdims=True)

    @pl.when(s == 0)
    def _():
        diag = jnp.exp(2.0 * bs - (D ** 0.5) * 1e-6)
        z1_ref[...] = -0.5 * jnp.sum(diag, keepdims=True)

    z1_ref[...] += part


_pair_call = pl.pallas_call(
    _pair_body,
    grid=(SP // TB,),
    in_specs=[
        pl.BlockSpec((D, SP), lambda s: (0, 0)),
        pl.BlockSpec((1, SP), lambda s: (0, 0)),
        pl.BlockSpec((D, TB), lambda s: (0, s)),
        pl.BlockSpec((1, TB), lambda s: (0, s)),
    ],
    out_specs=[
        pl.BlockSpec((1, 1), lambda s: (0, 0)),
    ],
    out_shape=[
        jax.ShapeDtypeStruct((1, 1), jnp.float32),
    ],
)


def _edge_body(p_hbm, beta_hbm, ii_hbm, jj_hbm, vc_hbm, out_hbm,
               p_v, b_v, ii_v, jj_v, vc_v, acc_v, sem):
    wid = lax.axis_index("s") * 2 + lax.axis_index("c")
    base = wid * CH
    copies = [
        pltpu.async_copy(p_hbm, p_v, sem),
        pltpu.async_copy(beta_hbm, b_v, sem),
        pltpu.async_copy(ii_hbm.at[pl.ds(base, CH)], ii_v, sem),
        pltpu.async_copy(jj_hbm.at[pl.ds(base, CH)], jj_v, sem),
        pltpu.async_copy(vc_hbm.at[pl.ds(base, CH)], vc_v, sem),
    ]
    for c in copies:
        c.wait()

    eps = jnp.full((32,), 1e-6, jnp.bfloat16)

    def body(t, acc):
        off = t * 16
        iv = ii_v[pl.ds(off, 16)]
        jv = jj_v[pl.ds(off, 16)]
        vc = vc_v[pl.ds(off, 16)]
        bi = plsc.load_gather(b_v, [iv])
        bj = plsc.load_gather(b_v, [jv])
        sqb = jnp.zeros((32,), jnp.bfloat16)
        for r in range(D // 2):
            row = jnp.full((16,), r, jnp.int32)
            wi = plsc.bitcast(plsc.load_gather(p_v, [row, iv]), jnp.bfloat16)
            wj = plsc.bitcast(plsc.load_gather(p_v, [row, jv]), jnp.bfloat16)
            t0 = wi - wj + eps
            sqb = sqb + t0 * t0
        slo, shi = plsc.unpack(sqb, format=plsc.PackFormat.INTERLEAVED)
        xc = jnp.maximum(slo + shi, 1e-30)
        yi = jnp.int32(0x5F3759DF) - (plsc.bitcast(xc, jnp.int32) >> 1)
        y = plsc.bitcast(yi, jnp.float32)
        y = y * (1.5 - 0.5 * xc * y * y)
        dist = xc * y
        return acc + vc * ((bi + bj) - dist)

    acc = lax.fori_loop(0, CH // 16, body, jnp.zeros((16,), jnp.float32),
                        unroll=2)
    acc_v[...] = acc
    pltpu.sync_copy(acc_v, out_hbm.at[wid])


@functools.cache
def _edge_call():
    return functools.partial(
        pl.kernel,
        out_type=jax.ShapeDtypeStruct((NSC, 16), jnp.float32),
        mesh=plsc.VectorSubcoreMesh(core_axis_name="c", subcore_axis_name="s"),
        scratch_types=[
            pltpu.VMEM((D // 2, N), jnp.int32),
            pltpu.VMEM((N,), jnp.float32),
            pltpu.VMEM((CH,), jnp.int32),
            pltpu.VMEM((CH,), jnp.int32),
            pltpu.VMEM((CH,), jnp.float32),
            pltpu.VMEM((16,), jnp.float32),
            pltpu.SemaphoreType.DMA,
        ],
        **_SC_PARAMS,
    )(_edge_body)


def kernel(beta, A, Z, Gate, sample_idx, sparse_sample_i, sparse_sample_j,
           valueC):
    zf = Z.astype(jnp.float32)
    gf = Gate.astype(jnp.float32)
    gtf = Gate.T.astype(jnp.float32)
    bf = beta.astype(jnp.float32)
    sidxp = jnp.pad(sample_idx.astype(jnp.int32), (0, SP - S))

    sg, gsamp = _sgather_call()(zf, gf, bf, sidxp)
    x, bsm, p = _dense_call(zf, gtf, A.astype(jnp.float32), sg, gsamp)
    partials = _edge_call()(p, bf, sparse_sample_i.astype(jnp.int32),
                            sparse_sample_j.astype(jnp.int32),
                            valueC.astype(jnp.float32))
    (z1,) = _pair_call(x, bsm, x, bsm)
    return jnp.sum(partials) - z1[0, 0]


# speedup vs baseline: 1.1778x; 1.1152x over previous
import functools

import jax
import jax.numpy as jnp
from jax import lax
from jax.experimental import pallas as pl
from jax.experimental.pallas import tpu as pltpu
from jax.experimental.pallas import tpu_sc as plsc

N = 10000
K = 16
D = 8
S = 2000
E = 320000

SP = 2048
TB = 512
NSC = 32
CH = E // NSC
NT = 2 * K + 1

_HI = jax.lax.Precision.HIGHEST

_SC_PARAMS = dict(
    compiler_params=pltpu.CompilerParams(use_tc_tiling_on_sc=False,
                                         needs_layout_passes=False),
)


_HF = SP // 2


def _sgather_body(z_hbm, gt_hbm, b_hbm, sidx_hbm, sg_hbm,
                  row_v, idx_v, out_v, out2_v):
    wid = lax.axis_index("s") * 2 + lax.axis_index("c")
    pltpu.sync_copy(sidx_hbm, idx_v)

    @pl.when(wid < K)
    def _():
        pltpu.sync_copy(z_hbm.at[wid], row_v)

    @pl.when(wid >= K)
    def _():
        pltpu.sync_copy(gt_hbm.at[wid - K], row_v)

    def body(i, _):
        iv = idx_v[pl.ds(i * 16, 16)]
        out_v[pl.ds(i * 16, 16)] = plsc.load_gather(row_v, [iv])
        return 0

    lax.fori_loop(0, SP // 16, body, 0)
    pltpu.sync_copy(out_v, sg_hbm.at[wid])

    @pl.when(wid < 2)
    def _():
        pltpu.sync_copy(b_hbm, row_v)

        def body2(i, _):
            iv = idx_v[pl.ds(wid * _HF + i * 16, 16)]
            out2_v[pl.ds(i * 16, 16)] = plsc.load_gather(row_v, [iv])
            return 0

        lax.fori_loop(0, _HF // 16, body2, 0)
        pltpu.sync_copy(out2_v, sg_hbm.at[NT - 1, pl.ds(wid * _HF, _HF)])


@functools.cache
def _sgather_call():
    return functools.partial(
        pl.kernel,
        out_type=jax.ShapeDtypeStruct((NT, SP), jnp.float32),
        mesh=plsc.VectorSubcoreMesh(core_axis_name="c", subcore_axis_name="s"),
        scratch_types=[
            pltpu.VMEM((N,), jnp.float32),
            pltpu.VMEM((SP,), jnp.int32),
            pltpu.VMEM((SP,), jnp.float32),
            pltpu.VMEM((_HF,), jnp.float32),
        ],
        **_SC_PARAMS,
    )(_sgather_body)


def _pack_rows(p):
    pb = p.astype(jnp.bfloat16)
    lo = lax.convert_element_type(
        lax.bitcast_convert_type(pb[0:D // 2, :], jnp.uint16), jnp.uint32)
    hi = lax.convert_element_type(
        lax.bitcast_convert_type(pb[D // 2:D, :], jnp.uint16), jnp.uint32)
    return lax.bitcast_convert_type(lo | (hi << 16), jnp.int32)


def _dense_body(z_ref, gt_ref, a_ref, sg_ref, x_ref, bsm_ref, p_ref):
    z = z_ref[...]
    zmax = jnp.max(z, axis=0, keepdims=True)
    ez = jnp.exp(z - zmax)
    zs = ez / jnp.sum(ez, axis=0, keepdims=True)
    g = 1.0 / (1.0 + jnp.exp(-gt_ref[...]))
    ztgt = zs * g
    ones_row = jnp.ones((1, N), jnp.float32)
    colsum = lax.dot_general(ones_row, ztgt, (((1,), (1,)), ((), ())),
                             preferred_element_type=jnp.float32, precision=_HI)

    sg = sg_ref[...]
    zraw = sg[0:K, :]
    graw = sg[K:2 * K, :]
    braw = sg[2 * K:2 * K + 1, :]
    zm = jnp.max(zraw, axis=0, keepdims=True)
    es = jnp.exp(zraw - zm)
    zsamp = es / jnp.sum(es, axis=0, keepdims=True)
    gs = 1.0 / (1.0 + jnp.exp(-graw))

    lane = lax.broadcasted_iota(jnp.int32, (1, SP), 1)
    valid = lane < S
    ztgsampt = jnp.where(valid, zsamp * gs, 0.0)
    bsm_ref[...] = jnp.where(valid, braw, -1e30)

    m = lax.dot_general(zsamp, ztgsampt, (((1,), (1,)), ((), ())),
                        preferred_element_type=jnp.float32, precision=_HI)
    m = m / colsum
    azc = lax.dot_general(a_ref[...], m, (((1,), (0,)), ((), ())),
                          preferred_element_type=jnp.float32, precision=_HI)
    x_ref[...] = lax.dot_general(azc, zsamp, (((1,), (0,)), ((), ())),
                                 preferred_element_type=jnp.float32,
                                 precision=_HI)
    p = lax.dot_general(azc, zs, (((1,), (0,)), ((), ())),
                        preferred_element_type=jnp.float32, precision=_HI)
    p_ref[...] = _pack_rows(p)


_dense_call = pl.pallas_call(
    _dense_body,
    out_shape=[
        jax.ShapeDtypeStruct((D, SP), jnp.float32),
        jax.ShapeDtypeStruct((1, SP), jnp.float32),
        jax.ShapeDtypeStruct((D // 2, N), jnp.int32),
    ],
)


def _pair_body(x_ref, bsm_ref, xt_ref, bst_ref, z1_ref):
    s = pl.program_id(0)
    x = x_ref[...]
    bs = bsm_ref[...]
    xi = xt_ref[...]
    bst = bst_ref[...]

    ycol = xi * (xi + 2e-6)
    ccol = lax.dot_general(ycol, jnp.ones((D, 1), jnp.float32),
                           (((0,), (0,)), ((), ())),
                           preferred_element_type=jnp.float32, precision=_HI)
    bcol = lax.dot_general(bst, jnp.ones((1, 1), jnp.float32),
                           (((0,), (0,)), ((), ())),
                           preferred_element_type=jnp.float32, precision=_HI)
    yrow = x * (x - 2e-6)
    rrow = lax.dot_general(jnp.ones((1, D), jnp.float32), yrow,
                           (((1,), (0,)), ((), ())),
                           preferred_element_type=jnp.float32, precision=_HI)
    cross = lax.dot_general(xi, x, (((0,), (0,)), ((), ())),
                            preferred_element_type=jnp.float32)
    d2 = jnp.maximum(ccol + rrow - 2.0 * cross + (D * 1e-12), 1e-30)
    dist = d2 * lax.rsqrt(d2)
    mat = jnp.exp(bcol + bs - dist)
    part = 0.5 * jnp.sum(mat, keepdims=True)

    @pl.when(s == 0)
    def _():
        diag = jnp.exp(2.0 * bs - (D ** 0.5) * 1e-6)
        z1_ref[...] = -0.5 * jnp.sum(diag, keepdims=True)

    z1_ref[...] += part


_pair_call = pl.pallas_call(
    _pair_body,
    grid=(SP // TB,),
    in_specs=[
        pl.BlockSpec((D, SP), lambda s: (0, 0)),
        pl.BlockSpec((1, SP), lambda s: (0, 0)),
        pl.BlockSpec((D, TB), lambda s: (0, s)),
        pl.BlockSpec((1, TB), lambda s: (0, s)),
    ],
    out_specs=[
        pl.BlockSpec((1, 1), lambda s: (0, 0)),
    ],
    out_shape=[
        jax.ShapeDtypeStruct((1, 1), jnp.float32),
    ],
)


def _edge_body(p_hbm, beta_hbm, ii_hbm, jj_hbm, vc_hbm, out_hbm,
               p_v, b_v, ii_v, jj_v, vc_v, acc_v, sem):
    wid = lax.axis_index("s") * 2 + lax.axis_index("c")
    base = wid * CH
    copies = [
        pltpu.async_copy(p_hbm, p_v, sem),
        pltpu.async_copy(beta_hbm, b_v, sem),
        pltpu.async_copy(ii_hbm.at[pl.ds(base, CH)], ii_v, sem),
        pltpu.async_copy(jj_hbm.at[pl.ds(base, CH)], jj_v, sem),
        pltpu.async_copy(vc_hbm.at[pl.ds(base, CH)], vc_v, sem),
    ]
    for c in copies:
        c.wait()

    eps = jnp.full((32,), 1e-6, jnp.bfloat16)

    def body(t, acc):
        off = t * 16
        iv = ii_v[pl.ds(off, 16)]
        jv = jj_v[pl.ds(off, 16)]
        vc = vc_v[pl.ds(off, 16)]
        bi = plsc.load_gather(b_v, [iv])
        bj = plsc.load_gather(b_v, [jv])
        sqb = jnp.zeros((32,), jnp.bfloat16)
        for r in range(D // 2):
            row = jnp.full((16,), r, jnp.int32)
            wi = plsc.bitcast(plsc.load_gather(p_v, [row, iv]), jnp.bfloat16)
            wj = plsc.bitcast(plsc.load_gather(p_v, [row, jv]), jnp.bfloat16)
            t0 = wi - wj + eps
            sqb = sqb + t0 * t0
        slo, shi = plsc.unpack(sqb, format=plsc.PackFormat.INTERLEAVED)
        xc = jnp.maximum(slo + shi, 1e-30)
        yi = jnp.int32(0x5F3759DF) - (plsc.bitcast(xc, jnp.int32) >> 1)
        y = plsc.bitcast(yi, jnp.float32)
        y = y * (1.5 - 0.5 * xc * y * y)
        dist = xc * y
        return acc + vc * ((bi + bj) - dist)

    acc = lax.fori_loop(0, CH // 16, body, jnp.zeros((16,), jnp.float32),
                        unroll=2)
    acc_v[...] = acc
    pltpu.sync_copy(acc_v, out_hbm.at[wid])


@functools.cache
def _edge_call():
    return functools.partial(
        pl.kernel,
        out_type=jax.ShapeDtypeStruct((NSC, 16), jnp.float32),
        mesh=plsc.VectorSubcoreMesh(core_axis_name="c", subcore_axis_name="s"),
        scratch_types=[
            pltpu.VMEM((D // 2, N), jnp.int32),
            pltpu.VMEM((N,), jnp.float32),
            pltpu.VMEM((CH,), jnp.int32),
            pltpu.VMEM((CH,), jnp.int32),
            pltpu.VMEM((CH,), jnp.float32),
            pltpu.VMEM((16,), jnp.float32),
            pltpu.SemaphoreType.DMA,
        ],
        **_SC_PARAMS,
    )(_edge_body)


def kernel(beta, A, Z, Gate, sample_idx, sparse_sample_i, sparse_sample_j,
           valueC):
    zf = Z.astype(jnp.float32)
    gtf = Gate.T.astype(jnp.float32)
    bf = beta.astype(jnp.float32)
    sidxp = jnp.pad(sample_idx.astype(jnp.int32), (0, SP - S))

    sg = _sgather_call()(zf, gtf, bf, sidxp)
    x, bsm, p = _dense_call(zf, gtf, A.astype(jnp.float32), sg)
    partials = _edge_call()(p, bf, sparse_sample_i.astype(jnp.int32),
                            sparse_sample_j.astype(jnp.int32),
                            valueC.astype(jnp.float32))
    (z1,) = _pair_call(x, bsm, x, bsm)
    return jnp.sum(partials) - z1[0, 0]


# speedup vs baseline: 1.1968x; 1.0162x over previous
import functools

import jax
import jax.numpy as jnp
from jax import lax
from jax.experimental import pallas as pl
from jax.experimental.pallas import tpu as pltpu
from jax.experimental.pallas import tpu_sc as plsc

N = 10000
K = 16
D = 8
S = 2000
E = 320000

SP = 2048
TB = 512
NSC = 32
CH = E // NSC
NT = 2 * K + 1

_HI = jax.lax.Precision.HIGHEST

_SC_PARAMS = dict(
    compiler_params=pltpu.CompilerParams(use_tc_tiling_on_sc=False,
                                         needs_layout_passes=False),
)


_HF = SP // 2


def _sgather_body(z_hbm, gt_hbm, b_hbm, sidx_hbm, sg_hbm,
                  row_v, idx_v, out_v, out2_v, sem, sem2):
    wid = lax.axis_index("s") * 2 + lax.axis_index("c")
    cp_idx = pltpu.async_copy(sidx_hbm, idx_v, sem)

    @pl.when(wid < K)
    def _():
        pltpu.async_copy(z_hbm.at[wid], row_v, sem2)

    @pl.when(wid >= K)
    def _():
        pltpu.async_copy(gt_hbm.at[wid - K], row_v, sem2)

    cp_idx.wait()
    pltpu.make_async_copy(z_hbm.at[0], row_v, sem2).wait()

    def body(i, _):
        iv = idx_v[pl.ds(i * 16, 16)]
        out_v[pl.ds(i * 16, 16)] = plsc.load_gather(row_v, [iv])
        return 0

    lax.fori_loop(0, SP // 16, body, 0)
    pltpu.sync_copy(out_v, sg_hbm.at[wid])

    @pl.when(wid < 2)
    def _():
        pltpu.sync_copy(b_hbm, row_v)

        def body2(i, _):
            iv = idx_v[pl.ds(wid * _HF + i * 16, 16)]
            out2_v[pl.ds(i * 16, 16)] = plsc.load_gather(row_v, [iv])
            return 0

        lax.fori_loop(0, _HF // 16, body2, 0)
        pltpu.sync_copy(out2_v, sg_hbm.at[NT - 1, pl.ds(wid * _HF, _HF)])


@functools.cache
def _sgather_call():
    return functools.partial(
        pl.kernel,
        out_type=jax.ShapeDtypeStruct((NT, SP), jnp.float32),
        mesh=plsc.VectorSubcoreMesh(core_axis_name="c", subcore_axis_name="s"),
        scratch_types=[
            pltpu.VMEM((N,), jnp.float32),
            pltpu.VMEM((SP,), jnp.int32),
            pltpu.VMEM((SP,), jnp.float32),
            pltpu.VMEM((_HF,), jnp.float32),
            pltpu.SemaphoreType.DMA,
            pltpu.SemaphoreType.DMA,
        ],
        **_SC_PARAMS,
    )(_sgather_body)


def _pack_rows(p):
    pb = p.astype(jnp.bfloat16)
    lo = lax.convert_element_type(
        lax.bitcast_convert_type(pb[0:D // 2, :], jnp.uint16), jnp.uint32)
    hi = lax.convert_element_type(
        lax.bitcast_convert_type(pb[D // 2:D, :], jnp.uint16), jnp.uint32)
    return lax.bitcast_convert_type(lo | (hi << 16), jnp.int32)


def _dense_body(z_ref, gt_ref, a_ref, sg_ref, x_ref, bsm_ref, p_ref):
    z = z_ref[...]
    zmax = jnp.max(z, axis=0, keepdims=True)
    ez = jnp.exp(z - zmax)
    zs = ez / jnp.sum(ez, axis=0, keepdims=True)
    g = 1.0 / (1.0 + jnp.exp(-gt_ref[...]))
    ztgt = zs * g
    ones_row = jnp.ones((1, N), jnp.float32)
    colsum = lax.dot_general(ones_row, ztgt, (((1,), (1,)), ((), ())),
                             preferred_element_type=jnp.float32, precision=_HI)

    sg = sg_ref[...]
    zraw = sg[0:K, :]
    graw = sg[K:2 * K, :]
    braw = sg[2 * K:2 * K + 1, :]
    zm = jnp.max(zraw, axis=0, keepdims=True)
    es = jnp.exp(zraw - zm)
    zsamp = es / jnp.sum(es, axis=0, keepdims=True)
    gs = 1.0 / (1.0 + jnp.exp(-graw))

    lane = lax.broadcasted_iota(jnp.int32, (1, SP), 1)
    valid = lane < S
    ztgsampt = jnp.where(valid, zsamp * gs, 0.0)
    bsm_ref[...] = jnp.where(valid, braw, -1e30)

    m = lax.dot_general(zsamp, ztgsampt, (((1,), (1,)), ((), ())),
                        preferred_element_type=jnp.float32, precision=_HI)
    m = m / colsum
    azc = lax.dot_general(a_ref[...], m, (((1,), (0,)), ((), ())),
                          preferred_element_type=jnp.float32, precision=_HI)
    x_ref[...] = lax.dot_general(azc, zsamp, (((1,), (0,)), ((), ())),
                                 preferred_element_type=jnp.float32,
                                 precision=_HI)
    p = lax.dot_general(azc, zs, (((1,), (0,)), ((), ())),
                        preferred_element_type=jnp.float32, precision=_HI)
    p_ref[...] = _pack_rows(p)


_dense_call = pl.pallas_call(
    _dense_body,
    out_shape=[
        jax.ShapeDtypeStruct((D, SP), jnp.float32),
        jax.ShapeDtypeStruct((1, SP), jnp.float32),
        jax.ShapeDtypeStruct((D // 2, N), jnp.int32),
    ],
)


def _pair_body(x_ref, bsm_ref, xt_ref, bst_ref, z1_ref):
    s = pl.program_id(0)
    x = x_ref[...]
    bs = bsm_ref[...]
    xi = xt_ref[...]
    bst = bst_ref[...]

    ycol = xi * (xi + 2e-6)
    ccol = lax.dot_general(ycol, jnp.ones((D, 1), jnp.float32),
                           (((0,), (0,)), ((), ())),
                           preferred_element_type=jnp.float32, precision=_HI)
    bcol = lax.dot_general(bst, jnp.ones((1, 1), jnp.float32),
                           (((0,), (0,)), ((), ())),
                           preferred_element_type=jnp.float32, precision=_HI)
    yrow = x * (x - 2e-6)
    rrow = lax.dot_general(jnp.ones((1, D), jnp.float32), yrow,
                           (((1,), (0,)), ((), ())),
                           preferred_element_type=jnp.float32, precision=_HI)
    cross = lax.dot_general(xi, x, (((0,), (0,)), ((), ())),
                            preferred_element_type=jnp.float32)
    d2 = jnp.maximum(ccol + rrow - 2.0 * cross + (D * 1e-12), 1e-30)
    dist = d2 * lax.rsqrt(d2)
    mat = jnp.exp(bcol + bs - dist)
    part = 0.5 * jnp.sum(mat, keepdims=True)

    @pl.when(s == 0)
    def _():
        diag = jnp.exp(2.0 * bs - (D ** 0.5) * 1e-6)
        z1_ref[...] = -0.5 * jnp.sum(diag, keepdims=True)

    z1_ref[...] += part


_pair_call = pl.pallas_call(
    _pair_body,
    grid=(SP // TB,),
    in_specs=[
        pl.BlockSpec((D, SP), lambda s: (0, 0)),
        pl.BlockSpec((1, SP), lambda s: (0, 0)),
        pl.BlockSpec((D, TB), lambda s: (0, s)),
        pl.BlockSpec((1, TB), lambda s: (0, s)),
    ],
    out_specs=[
        pl.BlockSpec((1, 1), lambda s: (0, 0)),
    ],
    out_shape=[
        jax.ShapeDtypeStruct((1, 1), jnp.float32),
    ],
)


def _edge_body(p_hbm, beta_hbm, ii_hbm, jj_hbm, vc_hbm, out_hbm,
               p_v, b_v, ii_v, jj_v, vc_v, acc_v, sem):
    wid = lax.axis_index("s") * 2 + lax.axis_index("c")
    base = wid * CH
    copies = [
        pltpu.async_copy(p_hbm, p_v, sem),
        pltpu.async_copy(beta_hbm, b_v, sem),
        pltpu.async_copy(ii_hbm.at[pl.ds(base, CH)], ii_v, sem),
        pltpu.async_copy(jj_hbm.at[pl.ds(base, CH)], jj_v, sem),
        pltpu.async_copy(vc_hbm.at[pl.ds(base, CH)], vc_v, sem),
    ]
    for c in copies:
        c.wait()

    eps = jnp.full((32,), 1e-6, jnp.bfloat16)

    def body(t, acc):
        off = t * 16
        iv = ii_v[pl.ds(off, 16)]
        jv = jj_v[pl.ds(off, 16)]
        vc = vc_v[pl.ds(off, 16)]
        bi = plsc.load_gather(b_v, [iv])
        bj = plsc.load_gather(b_v, [jv])
        sqb = jnp.zeros((32,), jnp.bfloat16)
        for r in range(D // 2):
            row = jnp.full((16,), r, jnp.int32)
            wi = plsc.bitcast(plsc.load_gather(p_v, [row, iv]), jnp.bfloat16)
            wj = plsc.bitcast(plsc.load_gather(p_v, [row, jv]), jnp.bfloat16)
            t0 = wi - wj + eps
            sqb = sqb + t0 * t0
        slo, shi = plsc.unpack(sqb, format=plsc.PackFormat.INTERLEAVED)
        xc = jnp.maximum(slo + shi, 1e-30)
        yi = jnp.int32(0x5F3759DF) - (plsc.bitcast(xc, jnp.int32) >> 1)
        y = plsc.bitcast(yi, jnp.float32)
        y = y * (1.5 - 0.5 * xc * y * y)
        dist = xc * y
        return acc + vc * ((bi + bj) - dist)

    acc = lax.fori_loop(0, CH // 16, body, jnp.zeros((16,), jnp.float32),
                        unroll=2)
    acc_v[...] = acc
    pltpu.sync_copy(acc_v, out_hbm.at[wid])


@functools.cache
def _edge_call():
    return functools.partial(
        pl.kernel,
        out_type=jax.ShapeDtypeStruct((NSC, 16), jnp.float32),
        mesh=plsc.VectorSubcoreMesh(core_axis_name="c", subcore_axis_name="s"),
        scratch_types=[
            pltpu.VMEM((D // 2, N), jnp.int32),
            pltpu.VMEM((N,), jnp.float32),
            pltpu.VMEM((CH,), jnp.int32),
            pltpu.VMEM((CH,), jnp.int32),
            pltpu.VMEM((CH,), jnp.float32),
            pltpu.VMEM((16,), jnp.float32),
            pltpu.SemaphoreType.DMA,
        ],
        **_SC_PARAMS,
    )(_edge_body)


def kernel(beta, A, Z, Gate, sample_idx, sparse_sample_i, sparse_sample_j,
           valueC):
    zf = Z.astype(jnp.float32)
    gtf = Gate.T.astype(jnp.float32)
    bf = beta.astype(jnp.float32)
    sidxp = jnp.pad(sample_idx.astype(jnp.int32), (0, SP - S))

    sg = _sgather_call()(zf, gtf, bf, sidxp)
    x, bsm, p = _dense_call(zf, gtf, A.astype(jnp.float32), sg)
    partials = _edge_call()(p, bf, sparse_sample_i.astype(jnp.int32),
                            sparse_sample_j.astype(jnp.int32),
                            valueC.astype(jnp.float32))
    (z1,) = _pair_call(x, bsm, x, bsm)
    return jnp.sum(partials) - z1[0, 0]
